# Initial kernel scaffold; baseline (speedup 1.0000x reference)
import jax
import jax.numpy as jnp
from jax.experimental import pallas as pl


def kernel(node_input, node_attr, edge_src, edge_dst, edge_attr, edge_scalars, batch, W_src, b_src, W_dst, b_dst, W_r0, b_r0, W_r1, b_r1, W_r2, b_r2, W_lin, b_lin, alpha_dot, W_proj, b_proj):
    raise NotImplementedError("write your pallas kernel here")


# speedup vs baseline: 3.5368x; 3.5368x over previous
import functools

import jax
import jax.numpy as jnp
from jax import lax
from jax.experimental import pallas as pl
from jax.experimental.pallas import tpu as pltpu
from jax.experimental.pallas import tpu_sc as plsc

N = 10000
E = 160000
D = 128
D_EDGE = 4
FC = 64
NUM_HEADS = 4
HEAD_DIM = 32
MUL_ALPHA = 32

NC = 2
NS = 16
NW = NC * NS
CHUNK = 40
ROWS_PER_W = E // NW
CPW = ROWS_PER_W // CHUNK
CPS = E // NS // CHUNK
NPAD = 10240

TE = 2000
GRID_E = E // TE

N_BLK = 1000
GRID_N = N // N_BLK


def _sigmoid(x):
    return 1.0 / (1.0 + jnp.exp(-x))


def _silu(x):
    return x * _sigmoid(x)


def _smooth_leaky_relu(x, a=0.2):
    return 0.5 * (1.0 + a) * x + 0.5 * (1.0 - a) * x * (2.0 * _sigmoid(x) - 1.0)


def _node_kernel(x_ref, attr_ref, ws_ref, bs_ref, wd_ref, bd_ref, os_ref, od_ref):
    x = x_ref[...] * attr_ref[...]
    os_ref[...] = jnp.dot(x, ws_ref[...], preferred_element_type=jnp.float32) + bs_ref[...]
    od_ref[...] = jnp.dot(x, wd_ref[...], preferred_element_type=jnp.float32) + bd_ref[...]


def _node_messages(node_input, node_attr, W_src, b_src, W_dst, b_dst):
    return pl.pallas_call(
        _node_kernel,
        grid=(GRID_N,),
        in_specs=[
            pl.BlockSpec((N_BLK, D), lambda i: (i, 0)),
            pl.BlockSpec((N_BLK, 1), lambda i: (i, 0)),
            pl.BlockSpec((D, D), lambda i: (0, 0)),
            pl.BlockSpec((1, D), lambda i: (0, 0)),
            pl.BlockSpec((D, D), lambda i: (0, 0)),
            pl.BlockSpec((1, D), lambda i: (0, 0)),
        ],
        out_specs=[
            pl.BlockSpec((N_BLK, D), lambda i: (i, 0)),
            pl.BlockSpec((N_BLK, D), lambda i: (i, 0)),
        ],
        out_shape=[
            jax.ShapeDtypeStruct((N, D), jnp.float32),
            jax.ShapeDtypeStruct((N, D), jnp.float32),
        ],
    )(node_input, node_attr, W_src, b_src.reshape(1, D), W_dst, b_dst.reshape(1, D))


def _gather_body(tbl_hbm, idx_hbm, out_hbm, idx_v, buf_v, sem):
    c = lax.axis_index("c")
    s = lax.axis_index("s")
    wid = c * NS + s
    pltpu.sync_copy(idx_hbm.at[wid], idx_v)

    def body(j, carry):
        pltpu.async_copy(tbl_hbm.at[idx_v.at[j]], buf_v, sem).wait()
        pltpu.sync_copy(buf_v, out_hbm.at[pl.ds((wid * CPW + j) * CHUNK, CHUNK)])
        return carry

    lax.fori_loop(0, CPW, body, 0)


_sc_gather = functools.partial(
    pl.kernel,
    out_type=jax.ShapeDtypeStruct((E, D), jnp.float32),
    mesh=plsc.VectorSubcoreMesh(core_axis_name="c", subcore_axis_name="s"),
    scratch_types=[
        pltpu.VMEM((CPW, CHUNK), jnp.int32),
        pltpu.VMEM((CHUNK, D), jnp.float32),
        pltpu.SemaphoreType.DMA,
    ],
)(_gather_body)


def _edge_kernel(gs_ref, gd_ref, sc_ref, ea_ref, wr0_ref, br0_ref, wr1_ref,
                 br1_ref, wr2_ref, br2_ref, wla_ref, wlv_ref, bla_ref, blv_ref,
                 a_ref, out_ref):
    h = _silu(jnp.dot(sc_ref[...], wr0_ref[...], preferred_element_type=jnp.float32)
              + br0_ref[...])
    h = _silu(jnp.dot(h, wr1_ref[...], preferred_element_type=jnp.float32)
              + br1_ref[...])
    msg = gs_ref[...] + gd_ref[...]
    ea = ea_ref[...]
    w2 = wr2_ref[...]
    b2 = br2_ref[...]
    wla = wla_ref[...]
    wlv = wlv_ref[...]
    f_a = jnp.broadcast_to(bla_ref[...], (TE, MUL_ALPHA))
    f_v = jnp.broadcast_to(blv_ref[...], (TE, D))
    for c in range(D_EDGE):
        w_c = (jnp.dot(h, w2[:, c * D:(c + 1) * D], preferred_element_type=jnp.float32)
               + b2[:, c * D:(c + 1) * D])
        d_c = msg * ea[:, c:c + 1] * w_c
        f_a = f_a + jnp.dot(d_c, wla[c * D:(c + 1) * D, :],
                            preferred_element_type=jnp.float32)
        f_v = f_v + jnp.dot(d_c, wlv[c * D:(c + 1) * D, :],
                            preferred_element_type=jnp.float32)
    alpha = _smooth_leaky_relu(f_a)
    logits = jnp.dot(alpha, a_ref[...], preferred_element_type=jnp.float32)
    ex = jnp.exp(logits)
    val = _silu(f_v)
    head = lax.broadcasted_iota(jnp.int32, (NUM_HEADS, D), 0)
    lane_head = lax.broadcasted_iota(jnp.int32, (NUM_HEADS, D), 1) // HEAD_DIM
    sel = (head == lane_head).astype(jnp.float32)
    exb = jnp.dot(ex, sel, preferred_element_type=jnp.float32)
    out_ref[0] = val * exb
    out_ref[1] = exb


def _edge_pipeline(g_src, g_dst, edge_scalars, edge_attr, W_r0, b_r0, W_r1,
                   b_r1, W_r2p, b_r2p, W_la, W_lv, b_la, b_lv, A):
    rep = lambda shape: pl.BlockSpec(shape, lambda i: tuple(0 for _ in shape))
    return pl.pallas_call(
        _edge_kernel,
        grid=(GRID_E,),
        in_specs=[
            pl.BlockSpec((TE, D), lambda i: (i, 0)),
            pl.BlockSpec((TE, D), lambda i: (i, 0)),
            pl.BlockSpec((TE, FC), lambda i: (i, 0)),
            pl.BlockSpec((TE, D_EDGE), lambda i: (i, 0)),
            rep((FC, FC)),
            rep((1, FC)),
            rep((FC, FC)),
            rep((1, FC)),
            rep((FC, D * D_EDGE)),
            rep((1, D * D_EDGE)),
            rep((D * D_EDGE, MUL_ALPHA)),
            rep((D * D_EDGE, D)),
            rep((1, MUL_ALPHA)),
            rep((1, D)),
            rep((MUL_ALPHA, NUM_HEADS)),
        ],
        out_specs=pl.BlockSpec((2, TE, D), lambda i: (0, i, 0)),
        out_shape=jax.ShapeDtypeStruct((2, E, D), jnp.float32),
    )(g_src, g_dst, edge_scalars, edge_attr, W_r0, b_r0.reshape(1, FC), W_r1,
      b_r1.reshape(1, FC), W_r2p, b_r2p.reshape(1, D * D_EDGE), W_la, W_lv,
      b_la.reshape(1, MUL_ALPHA), b_lv.reshape(1, D), A)


ROWS_PER_S = NPAD // NS


def _scatter_body(attn_hbm, idx_hbm, zeros_hbm, out_hbm, idx_v, buf_v, acc_sh):
    c = lax.axis_index("c")
    s = lax.axis_index("s")
    pltpu.sync_copy(zeros_hbm.at[pl.ds(s * ROWS_PER_S, ROWS_PER_S)],
                    acc_sh.at[pl.ds(s * ROWS_PER_S, ROWS_PER_S)])
    pltpu.sync_copy(idx_hbm.at[s], idx_v)
    plsc.subcore_barrier()

    def body(j, carry):
        pltpu.sync_copy(attn_hbm.at[c, pl.ds((s * CPS + j) * CHUNK, CHUNK)], buf_v)
        pltpu.sync_copy(buf_v, acc_sh.at[idx_v.at[j]], add=True)
        return carry

    lax.fori_loop(0, CPS, body, 0)
    plsc.subcore_barrier()
    pltpu.sync_copy(acc_sh.at[pl.ds(s * ROWS_PER_S, ROWS_PER_S)],
                    out_hbm.at[c, pl.ds(s * ROWS_PER_S, ROWS_PER_S)])


_sc_scatter = functools.partial(
    pl.kernel,
    out_type=jax.ShapeDtypeStruct((NC, NPAD, D), jnp.float32),
    mesh=plsc.VectorSubcoreMesh(core_axis_name="c", subcore_axis_name="s"),
    scratch_types=[
        pltpu.VMEM((CPS, CHUNK), jnp.int32),
        pltpu.VMEM((CHUNK, D), jnp.float32),
        pltpu.VMEM_SHARED((NPAD, D), jnp.float32),
    ],
)(_scatter_body)


def _proj_kernel(a0_ref, a1_ref, wp_ref, bp_ref, o_ref):
    num = a0_ref[0]
    den = a1_ref[0]
    x = num / (den + 1e-9)
    o_ref[...] = (jnp.dot(x, wp_ref[...], preferred_element_type=jnp.float32)
                  + bp_ref[...])


def _project(accum2, W_proj, b_proj):
    return pl.pallas_call(
        _proj_kernel,
        grid=(GRID_N,),
        in_specs=[
            pl.BlockSpec((1, N_BLK, D), lambda i: (0, i, 0)),
            pl.BlockSpec((1, N_BLK, D), lambda i: (1, i, 0)),
            pl.BlockSpec((D, D), lambda i: (0, 0)),
            pl.BlockSpec((1, D), lambda i: (0, 0)),
        ],
        out_specs=pl.BlockSpec((N_BLK, D), lambda i: (i, 0)),
        out_shape=jax.ShapeDtypeStruct((N, D), jnp.float32),
    )(accum2, accum2, W_proj, b_proj.reshape(1, D))


def kernel(node_input, node_attr, edge_src, edge_dst, edge_attr, edge_scalars,
           batch, W_src, b_src, W_dst, b_dst, W_r0, b_r0, W_r1, b_r1, W_r2,
           b_r2, W_lin, b_lin, alpha_dot, W_proj, b_proj):
    W_r2p = W_r2.reshape(FC, D, D_EDGE).transpose(0, 2, 1).reshape(FC, D * D_EDGE)
    b_r2p = b_r2.reshape(D, D_EDGE).T.reshape(D * D_EDGE)
    W_linp = W_lin.reshape(D, D_EDGE, MUL_ALPHA + D).transpose(1, 0, 2)
    W_linp = W_linp.reshape(D * D_EDGE, MUL_ALPHA + D)
    W_la = W_linp[:, :MUL_ALPHA]
    W_lv = W_linp[:, MUL_ALPHA:]
    b_la = b_lin[:MUL_ALPHA]
    b_lv = b_lin[MUL_ALPHA:]
    flat = alpha_dot.reshape(-1)
    rows = jnp.arange(MUL_ALPHA)
    A = jnp.zeros((MUL_ALPHA, NUM_HEADS), jnp.float32).at[rows, rows // 8].set(flat)

    src_idx = edge_src.astype(jnp.int32).reshape(NW, CPW, CHUNK)
    dst_idx = edge_dst.astype(jnp.int32).reshape(NW, CPW, CHUNK)
    dst_idx_sc = edge_dst.astype(jnp.int32).reshape(NS, CPS, CHUNK)

    msg_src, msg_dst = _node_messages(node_input, node_attr, W_src, b_src,
                                      W_dst, b_dst)
    g_src = _sc_gather(msg_src, src_idx)
    g_dst = _sc_gather(msg_dst, dst_idx)
    attn_ext = _edge_pipeline(g_src, g_dst, edge_scalars, edge_attr, W_r0,
                              b_r0, W_r1, b_r1, W_r2p, b_r2p, W_la, W_lv,
                              b_la, b_lv, A)
    zeros = jnp.zeros((NPAD, D), jnp.float32)
    accum2 = _sc_scatter(attn_ext, dst_idx_sc, zeros)
    return _project(accum2, W_proj, b_proj)


# speedup vs baseline: 4.3709x; 1.2358x over previous
import functools

import jax
import jax.numpy as jnp
from jax import lax
from jax.experimental import pallas as pl
from jax.experimental.pallas import tpu as pltpu
from jax.experimental.pallas import tpu_sc as plsc

N = 10000
E = 160000
D = 128
D_EDGE = 4
FC = 64
NUM_HEADS = 4
HEAD_DIM = 32
MUL_ALPHA = 32
DEN_W = 8

NC = 2
NS = 16
NW = NC * NS
CHUNK = 40
ROWS_PER_W = E // NW
CPW = ROWS_PER_W // CHUNK
QUADS = CPW // 4
NPAD = 10240
ROWS_PER_S = NPAD // NS

TE = 2000
GRID_E = E // TE

N_BLK = 1000
GRID_N = N // N_BLK


def _sigmoid(x):
    return 1.0 / (1.0 + jnp.exp(-x))


def _silu(x):
    return x * _sigmoid(x)


def _smooth_leaky_relu(x, a=0.2):
    return 0.5 * (1.0 + a) * x + 0.5 * (1.0 - a) * x * (2.0 * _sigmoid(x) - 1.0)


def _node_kernel(x_ref, attr_ref, ws_ref, bs_ref, wd_ref, bd_ref, os_ref, od_ref):
    x = x_ref[...] * attr_ref[...]
    os_ref[...] = jnp.dot(x, ws_ref[...], preferred_element_type=jnp.float32) + bs_ref[...]
    od_ref[...] = jnp.dot(x, wd_ref[...], preferred_element_type=jnp.float32) + bd_ref[...]


def _node_messages(node_input, node_attr, W_src, b_src, W_dst, b_dst):
    return pl.pallas_call(
        _node_kernel,
        grid=(GRID_N,),
        in_specs=[
            pl.BlockSpec((N_BLK, D), lambda i: (i, 0)),
            pl.BlockSpec((N_BLK, 1), lambda i: (i, 0)),
            pl.BlockSpec((D, D), lambda i: (0, 0)),
            pl.BlockSpec((1, D), lambda i: (0, 0)),
            pl.BlockSpec((D, D), lambda i: (0, 0)),
            pl.BlockSpec((1, D), lambda i: (0, 0)),
        ],
        out_specs=[
            pl.BlockSpec((N_BLK, D), lambda i: (i, 0)),
            pl.BlockSpec((N_BLK, D), lambda i: (i, 0)),
        ],
        out_shape=[
            jax.ShapeDtypeStruct((N, D), jnp.float32),
            jax.ShapeDtypeStruct((N, D), jnp.float32),
        ],
    )(node_input, node_attr, W_src, b_src.reshape(1, D), W_dst, b_dst.reshape(1, D))


def _gather_body(tbl_hbm, idx_hbm, out_hbm, idx_v, b0, b1, b2, b3,
                 s0, s1, s2, s3):
    c = lax.axis_index("c")
    s = lax.axis_index("s")
    wid = c * NS + s
    pltpu.sync_copy(idx_hbm.at[wid], idx_v)
    base = wid * CPW

    def quad(q, carry):
        g = q * 4
        cp0 = pltpu.async_copy(tbl_hbm.at[idx_v.at[g]], b0, s0)
        cp1 = pltpu.async_copy(tbl_hbm.at[idx_v.at[g + 1]], b1, s1)
        cp2 = pltpu.async_copy(tbl_hbm.at[idx_v.at[g + 2]], b2, s2)
        cp3 = pltpu.async_copy(tbl_hbm.at[idx_v.at[g + 3]], b3, s3)
        cp0.wait()
        pltpu.sync_copy(b0, out_hbm.at[pl.ds((base + g) * CHUNK, CHUNK)])
        cp1.wait()
        pltpu.sync_copy(b1, out_hbm.at[pl.ds((base + g + 1) * CHUNK, CHUNK)])
        cp2.wait()
        pltpu.sync_copy(b2, out_hbm.at[pl.ds((base + g + 2) * CHUNK, CHUNK)])
        cp3.wait()
        pltpu.sync_copy(b3, out_hbm.at[pl.ds((base + g + 3) * CHUNK, CHUNK)])
        return carry

    lax.fori_loop(0, QUADS, quad, 0)
    g = QUADS * 4
    pltpu.async_copy(tbl_hbm.at[idx_v.at[g]], b0, s0).wait()
    pltpu.sync_copy(b0, out_hbm.at[pl.ds((base + g) * CHUNK, CHUNK)])


_sc_gather = functools.partial(
    pl.kernel,
    out_type=jax.ShapeDtypeStruct((E, D), jnp.float32),
    mesh=plsc.VectorSubcoreMesh(core_axis_name="c", subcore_axis_name="s"),
    scratch_types=[
        pltpu.VMEM((CPW, CHUNK), jnp.int32),
        pltpu.VMEM((CHUNK, D), jnp.float32),
        pltpu.VMEM((CHUNK, D), jnp.float32),
        pltpu.VMEM((CHUNK, D), jnp.float32),
        pltpu.VMEM((CHUNK, D), jnp.float32),
        pltpu.SemaphoreType.DMA,
        pltpu.SemaphoreType.DMA,
        pltpu.SemaphoreType.DMA,
        pltpu.SemaphoreType.DMA,
    ],
)(_gather_body)


def _edge_kernel(gs_ref, gd_ref, sc_ref, ea_ref, wr0_ref, br0_ref, wr1_ref,
                 br1_ref, wr2_ref, br2_ref, wla_ref, wlv_ref, bla_ref, blv_ref,
                 a_ref, out_ref):
    h = _silu(jnp.dot(sc_ref[...], wr0_ref[...], preferred_element_type=jnp.float32)
              + br0_ref[...])
    h = _silu(jnp.dot(h, wr1_ref[...], preferred_element_type=jnp.float32)
              + br1_ref[...])
    msg = gs_ref[...] + gd_ref[...]
    ea = ea_ref[...]
    w2 = wr2_ref[...]
    b2 = br2_ref[...]
    wla = wla_ref[...]
    wlv = wlv_ref[...]
    f_a = jnp.broadcast_to(bla_ref[...], (TE, MUL_ALPHA))
    f_v = jnp.broadcast_to(blv_ref[...], (TE, D))
    for c in range(D_EDGE):
        w_c = (jnp.dot(h, w2[:, c * D:(c + 1) * D], preferred_element_type=jnp.float32)
               + b2[:, c * D:(c + 1) * D])
        d_c = msg * ea[:, c:c + 1] * w_c
        f_a = f_a + jnp.dot(d_c, wla[c * D:(c + 1) * D, :],
                            preferred_element_type=jnp.float32)
        f_v = f_v + jnp.dot(d_c, wlv[c * D:(c + 1) * D, :],
                            preferred_element_type=jnp.float32)
    alpha = _smooth_leaky_relu(f_a)
    logits = jnp.dot(alpha, a_ref[...], preferred_element_type=jnp.float32)
    ex = jnp.exp(logits)
    val = _silu(f_v)
    head = lax.broadcasted_iota(jnp.int32, (NUM_HEADS, D), 0)
    lane_head = lax.broadcasted_iota(jnp.int32, (NUM_HEADS, D), 1) // HEAD_DIM
    sel = (head == lane_head).astype(jnp.float32)
    exb = jnp.dot(ex, sel, preferred_element_type=jnp.float32)
    out_ref[0] = val * exb
    out_ref[1] = exb


def _edge_pipeline(g_src, g_dst, edge_scalars, edge_attr, W_r0, b_r0, W_r1,
                   b_r1, W_r2p, b_r2p, W_la, W_lv, b_la, b_lv, A):
    rep = lambda shape: pl.BlockSpec(shape, lambda i: tuple(0 for _ in shape))
    return pl.pallas_call(
        _edge_kernel,
        grid=(GRID_E,),
        in_specs=[
            pl.BlockSpec((TE, D), lambda i: (i, 0)),
            pl.BlockSpec((TE, D), lambda i: (i, 0)),
            pl.BlockSpec((TE, FC), lambda i: (i, 0)),
            pl.BlockSpec((TE, D_EDGE), lambda i: (i, 0)),
            rep((FC, FC)),
            rep((1, FC)),
            rep((FC, FC)),
            rep((1, FC)),
            rep((FC, D * D_EDGE)),
            rep((1, D * D_EDGE)),
            rep((D * D_EDGE, MUL_ALPHA)),
            rep((D * D_EDGE, D)),
            rep((1, MUL_ALPHA)),
            rep((1, D)),
            rep((MUL_ALPHA, NUM_HEADS)),
        ],
        out_specs=pl.BlockSpec((2, TE, D), lambda i: (0, i, 0)),
        out_shape=jax.ShapeDtypeStruct((2, E, D), jnp.float32),
    )(g_src, g_dst, edge_scalars, edge_attr, W_r0, b_r0.reshape(1, FC), W_r1,
      b_r1.reshape(1, FC), W_r2p, b_r2p.reshape(1, D * D_EDGE), W_la, W_lv,
      b_la.reshape(1, MUL_ALPHA), b_lv.reshape(1, D), A)


CPS = E // NS // CHUNK


def _scatter_body(attn_hbm, idx_hbm, zeros_hbm, out_hbm, idx_v, buf_v, acc_sh):
    c = lax.axis_index("c")
    s = lax.axis_index("s")
    pltpu.sync_copy(zeros_hbm.at[pl.ds(s * ROWS_PER_S, ROWS_PER_S)],
                    acc_sh.at[pl.ds(s * ROWS_PER_S, ROWS_PER_S)])
    pltpu.sync_copy(idx_hbm.at[s], idx_v)
    plsc.subcore_barrier()

    def body(j, carry):
        pltpu.sync_copy(attn_hbm.at[c, pl.ds((s * CPS + j) * CHUNK, CHUNK)], buf_v)
        pltpu.sync_copy(buf_v, acc_sh.at[idx_v.at[j]], add=True)
        return carry

    lax.fori_loop(0, CPS, body, 0)
    plsc.subcore_barrier()
    pltpu.sync_copy(acc_sh.at[pl.ds(s * ROWS_PER_S, ROWS_PER_S)],
                    out_hbm.at[c, pl.ds(s * ROWS_PER_S, ROWS_PER_S)])


_sc_scatter = functools.partial(
    pl.kernel,
    out_type=jax.ShapeDtypeStruct((NC, NPAD, D), jnp.float32),
    mesh=plsc.VectorSubcoreMesh(core_axis_name="c", subcore_axis_name="s"),
    scratch_types=[
        pltpu.VMEM((CPS, CHUNK), jnp.int32),
        pltpu.VMEM((CHUNK, D), jnp.float32),
        pltpu.VMEM_SHARED((NPAD, D), jnp.float32),
    ],
)(_scatter_body)


def _proj_kernel(a0_ref, a1_ref, wp_ref, bp_ref, o_ref):
    num = a0_ref[0]
    den = a1_ref[0]
    x = num / (den + 1e-9)
    o_ref[...] = (jnp.dot(x, wp_ref[...], preferred_element_type=jnp.float32)
                  + bp_ref[...])


def _project(accum2, W_proj, b_proj):
    return pl.pallas_call(
        _proj_kernel,
        grid=(GRID_N,),
        in_specs=[
            pl.BlockSpec((1, N_BLK, D), lambda i: (0, i, 0)),
            pl.BlockSpec((1, N_BLK, D), lambda i: (1, i, 0)),
            pl.BlockSpec((D, D), lambda i: (0, 0)),
            pl.BlockSpec((1, D), lambda i: (0, 0)),
        ],
        out_specs=pl.BlockSpec((N_BLK, D), lambda i: (i, 0)),
        out_shape=jax.ShapeDtypeStruct((N, D), jnp.float32),
    )(accum2, accum2, W_proj, b_proj.reshape(1, D))


def kernel(node_input, node_attr, edge_src, edge_dst, edge_attr, edge_scalars,
           batch, W_src, b_src, W_dst, b_dst, W_r0, b_r0, W_r1, b_r1, W_r2,
           b_r2, W_lin, b_lin, alpha_dot, W_proj, b_proj):
    W_r2p = W_r2.reshape(FC, D, D_EDGE).transpose(0, 2, 1).reshape(FC, D * D_EDGE)
    b_r2p = b_r2.reshape(D, D_EDGE).T.reshape(D * D_EDGE)
    W_linp = W_lin.reshape(D, D_EDGE, MUL_ALPHA + D).transpose(1, 0, 2)
    W_linp = W_linp.reshape(D * D_EDGE, MUL_ALPHA + D)
    W_la = W_linp[:, :MUL_ALPHA]
    W_lv = W_linp[:, MUL_ALPHA:]
    b_la = b_lin[:MUL_ALPHA]
    b_lv = b_lin[MUL_ALPHA:]
    flat = alpha_dot.reshape(-1)
    rows = jnp.arange(MUL_ALPHA)
    A = jnp.zeros((MUL_ALPHA, NUM_HEADS), jnp.float32).at[rows, rows // 8].set(flat)

    src_idx = edge_src.astype(jnp.int32).reshape(NW, CPW, CHUNK)
    dst_idx = edge_dst.astype(jnp.int32).reshape(NW, CPW, CHUNK)
    dst_idx_sc = edge_dst.astype(jnp.int32).reshape(NS, CPS, CHUNK)

    msg_src, msg_dst = _node_messages(node_input, node_attr, W_src, b_src,
                                      W_dst, b_dst)
    g_src = _sc_gather(msg_src, src_idx)
    g_dst = _sc_gather(msg_dst, dst_idx)
    attn_ext = _edge_pipeline(g_src, g_dst, edge_scalars, edge_attr, W_r0,
                              b_r0, W_r1, b_r1, W_r2p, b_r2p, W_la, W_lv,
                              b_la, b_lv, A)
    zeros = jnp.zeros((NPAD, D), jnp.float32)
    accum2 = _sc_scatter(attn_ext, dst_idx_sc, zeros)
    return _project(accum2, W_proj, b_proj)


# speedup vs baseline: 4.8813x; 1.1168x over previous
import functools

import jax
import jax.numpy as jnp
from jax import lax
from jax.experimental import pallas as pl
from jax.experimental.pallas import tpu as pltpu
from jax.experimental.pallas import tpu_sc as plsc

N = 10000
E = 160000
D = 128
D_EDGE = 4
FC = 64
NUM_HEADS = 4
HEAD_DIM = 32
MUL_ALPHA = 32
DEN_W = 8

NC = 2
NS = 16
NW = NC * NS
CHUNK = 40
ROWS_PER_W = E // NW
CPW = ROWS_PER_W // CHUNK
QUADS = CPW // 4
NPAD = 10240
ROWS_PER_S = NPAD // NS

TE = 2000
GRID_E = E // TE

N_BLK = 1000
GRID_N = N // N_BLK


def _sigmoid(x):
    return 1.0 / (1.0 + jnp.exp(-x))


def _silu(x):
    return x * _sigmoid(x)


def _smooth_leaky_relu(x, a=0.2):
    return 0.5 * (1.0 + a) * x + 0.5 * (1.0 - a) * x * (2.0 * _sigmoid(x) - 1.0)


def _node_kernel(x_ref, attr_ref, ws_ref, bs_ref, wd_ref, bd_ref, os_ref, od_ref):
    x = x_ref[...] * attr_ref[...]
    os_ref[...] = jnp.dot(x, ws_ref[...], preferred_element_type=jnp.float32) + bs_ref[...]
    od_ref[...] = jnp.dot(x, wd_ref[...], preferred_element_type=jnp.float32) + bd_ref[...]


def _node_messages(node_input, node_attr, W_src, b_src, W_dst, b_dst):
    return pl.pallas_call(
        _node_kernel,
        grid=(GRID_N,),
        in_specs=[
            pl.BlockSpec((N_BLK, D), lambda i: (i, 0)),
            pl.BlockSpec((N_BLK, 1), lambda i: (i, 0)),
            pl.BlockSpec((D, D), lambda i: (0, 0)),
            pl.BlockSpec((1, D), lambda i: (0, 0)),
            pl.BlockSpec((D, D), lambda i: (0, 0)),
            pl.BlockSpec((1, D), lambda i: (0, 0)),
        ],
        out_specs=[
            pl.BlockSpec((N_BLK, D), lambda i: (i, 0)),
            pl.BlockSpec((N_BLK, D), lambda i: (i, 0)),
        ],
        out_shape=[
            jax.ShapeDtypeStruct((N, D), jnp.float32),
            jax.ShapeDtypeStruct((N, D), jnp.float32),
        ],
    )(node_input, node_attr, W_src, b_src.reshape(1, D), W_dst, b_dst.reshape(1, D))


def _gather_body(tbl_hbm, idx_hbm, out_hbm, idx_v, b0, b1, b2, b3,
                 s0, s1, s2, s3):
    c = lax.axis_index("c")
    s = lax.axis_index("s")
    wid = c * NS + s
    pltpu.sync_copy(idx_hbm.at[wid], idx_v)
    base = wid * CPW

    def quad(q, carry):
        g = q * 4
        cp0 = pltpu.async_copy(tbl_hbm.at[idx_v.at[g]], b0, s0)
        cp1 = pltpu.async_copy(tbl_hbm.at[idx_v.at[g + 1]], b1, s1)
        cp2 = pltpu.async_copy(tbl_hbm.at[idx_v.at[g + 2]], b2, s2)
        cp3 = pltpu.async_copy(tbl_hbm.at[idx_v.at[g + 3]], b3, s3)
        cp0.wait()
        pltpu.sync_copy(b0, out_hbm.at[pl.ds((base + g) * CHUNK, CHUNK)])
        cp1.wait()
        pltpu.sync_copy(b1, out_hbm.at[pl.ds((base + g + 1) * CHUNK, CHUNK)])
        cp2.wait()
        pltpu.sync_copy(b2, out_hbm.at[pl.ds((base + g + 2) * CHUNK, CHUNK)])
        cp3.wait()
        pltpu.sync_copy(b3, out_hbm.at[pl.ds((base + g + 3) * CHUNK, CHUNK)])
        return carry

    lax.fori_loop(0, QUADS, quad, 0)
    g = QUADS * 4
    pltpu.async_copy(tbl_hbm.at[idx_v.at[g]], b0, s0).wait()
    pltpu.sync_copy(b0, out_hbm.at[pl.ds((base + g) * CHUNK, CHUNK)])


_sc_gather = functools.partial(
    pl.kernel,
    out_type=jax.ShapeDtypeStruct((E, D), jnp.float32),
    mesh=plsc.VectorSubcoreMesh(core_axis_name="c", subcore_axis_name="s"),
    scratch_types=[
        pltpu.VMEM((CPW, CHUNK), jnp.int32),
        pltpu.VMEM((CHUNK, D), jnp.float32),
        pltpu.VMEM((CHUNK, D), jnp.float32),
        pltpu.VMEM((CHUNK, D), jnp.float32),
        pltpu.VMEM((CHUNK, D), jnp.float32),
        pltpu.SemaphoreType.DMA,
        pltpu.SemaphoreType.DMA,
        pltpu.SemaphoreType.DMA,
        pltpu.SemaphoreType.DMA,
    ],
)(_gather_body)


def _edge_kernel(gs_ref, gd_ref, sc_ref, ea_ref, wr0_ref, br0_ref, wr1_ref,
                 br1_ref, wr2_ref, br2_ref, wla_ref, wlv_ref, bla_ref, blv_ref,
                 a_ref, out_ref):
    h = _silu(jnp.dot(sc_ref[...], wr0_ref[...], preferred_element_type=jnp.float32)
              + br0_ref[...])
    h = _silu(jnp.dot(h, wr1_ref[...], preferred_element_type=jnp.float32)
              + br1_ref[...])
    msg = gs_ref[...] + gd_ref[...]
    ea = ea_ref[...]
    w2 = wr2_ref[...]
    b2 = br2_ref[...]
    wla = wla_ref[...]
    wlv = wlv_ref[...]
    f_a = jnp.broadcast_to(bla_ref[...], (TE, MUL_ALPHA))
    f_v = jnp.broadcast_to(blv_ref[...], (TE, D))
    for c in range(D_EDGE):
        w_c = (jnp.dot(h, w2[:, c * D:(c + 1) * D], preferred_element_type=jnp.float32)
               + b2[:, c * D:(c + 1) * D])
        d_c = msg * ea[:, c:c + 1] * w_c
        f_a = f_a + jnp.dot(d_c, wla[c * D:(c + 1) * D, :],
                            preferred_element_type=jnp.float32)
        f_v = f_v + jnp.dot(d_c, wlv[c * D:(c + 1) * D, :],
                            preferred_element_type=jnp.float32)
    alpha = _smooth_leaky_relu(f_a)
    logits = jnp.dot(alpha, a_ref[...], preferred_element_type=jnp.float32)
    ex = jnp.exp(logits)
    val = _silu(f_v)
    head = lax.broadcasted_iota(jnp.int32, (NUM_HEADS, D), 0)
    lane_head = lax.broadcasted_iota(jnp.int32, (NUM_HEADS, D), 1) // HEAD_DIM
    sel = (head == lane_head).astype(jnp.float32)
    exb = jnp.dot(ex, sel, preferred_element_type=jnp.float32)
    out_ref[0] = val * exb
    out_ref[1] = exb


def _edge_pipeline(g_src, g_dst, edge_scalars, edge_attr, W_r0, b_r0, W_r1,
                   b_r1, W_r2p, b_r2p, W_la, W_lv, b_la, b_lv, A):
    rep = lambda shape: pl.BlockSpec(shape, lambda i: tuple(0 for _ in shape))
    return pl.pallas_call(
        _edge_kernel,
        grid=(GRID_E,),
        in_specs=[
            pl.BlockSpec((TE, D), lambda i: (i, 0)),
            pl.BlockSpec((TE, D), lambda i: (i, 0)),
            pl.BlockSpec((TE, FC), lambda i: (i, 0)),
            pl.BlockSpec((TE, D_EDGE), lambda i: (i, 0)),
            rep((FC, FC)),
            rep((1, FC)),
            rep((FC, FC)),
            rep((1, FC)),
            rep((FC, D * D_EDGE)),
            rep((1, D * D_EDGE)),
            rep((D * D_EDGE, MUL_ALPHA)),
            rep((D * D_EDGE, D)),
            rep((1, MUL_ALPHA)),
            rep((1, D)),
            rep((MUL_ALPHA, NUM_HEADS)),
        ],
        out_specs=pl.BlockSpec((2, TE, D), lambda i: (0, i, 0)),
        out_shape=jax.ShapeDtypeStruct((2, E, D), jnp.float32),
    )(g_src, g_dst, edge_scalars, edge_attr, W_r0, b_r0.reshape(1, FC), W_r1,
      b_r1.reshape(1, FC), W_r2p, b_r2p.reshape(1, D * D_EDGE), W_la, W_lv,
      b_la.reshape(1, MUL_ALPHA), b_lv.reshape(1, D), A)


CPS = E // NS // CHUNK


def _scatter_body(attn_hbm, idx_hbm, zeros_hbm, out_hbm, idx_v, b0, b1,
                  s0, s1, acc_sh):
    c = lax.axis_index("c")
    s = lax.axis_index("s")
    pltpu.sync_copy(zeros_hbm.at[pl.ds(s * ROWS_PER_S, ROWS_PER_S)],
                    acc_sh.at[pl.ds(s * ROWS_PER_S, ROWS_PER_S)])
    pltpu.sync_copy(idx_hbm.at[s], idx_v)
    plsc.subcore_barrier()
    base = s * CPS

    def pair(p, carry):
        g = p * 2
        cp0 = pltpu.async_copy(attn_hbm.at[c, pl.ds((base + g) * CHUNK, CHUNK)],
                               b0, s0)
        cp1 = pltpu.async_copy(
            attn_hbm.at[c, pl.ds((base + g + 1) * CHUNK, CHUNK)], b1, s1)
        cp0.wait()
        pltpu.sync_copy(b0, acc_sh.at[idx_v.at[g]], add=True)
        cp1.wait()
        pltpu.sync_copy(b1, acc_sh.at[idx_v.at[g + 1]], add=True)
        return carry

    lax.fori_loop(0, CPS // 2, pair, 0)
    plsc.subcore_barrier()
    pltpu.sync_copy(acc_sh.at[pl.ds(s * ROWS_PER_S, ROWS_PER_S)],
                    out_hbm.at[c, pl.ds(s * ROWS_PER_S, ROWS_PER_S)])


_sc_scatter = functools.partial(
    pl.kernel,
    out_type=jax.ShapeDtypeStruct((NC, NPAD, D), jnp.float32),
    mesh=plsc.VectorSubcoreMesh(core_axis_name="c", subcore_axis_name="s"),
    scratch_types=[
        pltpu.VMEM((CPS, CHUNK), jnp.int32),
        pltpu.VMEM((CHUNK, D), jnp.float32),
        pltpu.VMEM((CHUNK, D), jnp.float32),
        pltpu.SemaphoreType.DMA,
        pltpu.SemaphoreType.DMA,
        pltpu.VMEM_SHARED((NPAD, D), jnp.float32),
    ],
)(_scatter_body)


def _proj_kernel(a0_ref, a1_ref, wp_ref, bp_ref, o_ref):
    num = a0_ref[0]
    den = a1_ref[0]
    x = num / (den + 1e-9)
    o_ref[...] = (jnp.dot(x, wp_ref[...], preferred_element_type=jnp.float32)
                  + bp_ref[...])


def _project(accum2, W_proj, b_proj):
    return pl.pallas_call(
        _proj_kernel,
        grid=(GRID_N,),
        in_specs=[
            pl.BlockSpec((1, N_BLK, D), lambda i: (0, i, 0)),
            pl.BlockSpec((1, N_BLK, D), lambda i: (1, i, 0)),
            pl.BlockSpec((D, D), lambda i: (0, 0)),
            pl.BlockSpec((1, D), lambda i: (0, 0)),
        ],
        out_specs=pl.BlockSpec((N_BLK, D), lambda i: (i, 0)),
        out_shape=jax.ShapeDtypeStruct((N, D), jnp.float32),
    )(accum2, accum2, W_proj, b_proj.reshape(1, D))


def kernel(node_input, node_attr, edge_src, edge_dst, edge_attr, edge_scalars,
           batch, W_src, b_src, W_dst, b_dst, W_r0, b_r0, W_r1, b_r1, W_r2,
           b_r2, W_lin, b_lin, alpha_dot, W_proj, b_proj):
    W_r2p = W_r2.reshape(FC, D, D_EDGE).transpose(0, 2, 1).reshape(FC, D * D_EDGE)
    b_r2p = b_r2.reshape(D, D_EDGE).T.reshape(D * D_EDGE)
    W_linp = W_lin.reshape(D, D_EDGE, MUL_ALPHA + D).transpose(1, 0, 2)
    W_linp = W_linp.reshape(D * D_EDGE, MUL_ALPHA + D)
    W_la = W_linp[:, :MUL_ALPHA]
    W_lv = W_linp[:, MUL_ALPHA:]
    b_la = b_lin[:MUL_ALPHA]
    b_lv = b_lin[MUL_ALPHA:]
    flat = alpha_dot.reshape(-1)
    rows = jnp.arange(MUL_ALPHA)
    A = jnp.zeros((MUL_ALPHA, NUM_HEADS), jnp.float32).at[rows, rows // 8].set(flat)

    src_idx = edge_src.astype(jnp.int32).reshape(NW, CPW, CHUNK)
    dst_idx = edge_dst.astype(jnp.int32).reshape(NW, CPW, CHUNK)
    dst_idx_sc = edge_dst.astype(jnp.int32).reshape(NS, CPS, CHUNK)

    msg_src, msg_dst = _node_messages(node_input, node_attr, W_src, b_src,
                                      W_dst, b_dst)
    g_src = _sc_gather(msg_src, src_idx)
    g_dst = _sc_gather(msg_dst, dst_idx)
    attn_ext = _edge_pipeline(g_src, g_dst, edge_scalars, edge_attr, W_r0,
                              b_r0, W_r1, b_r1, W_r2p, b_r2p, W_la, W_lv,
                              b_la, b_lv, A)
    zeros = jnp.zeros((NPAD, D), jnp.float32)
    accum2 = _sc_scatter(attn_ext, dst_idx_sc, zeros)
    return _project(accum2, W_proj, b_proj)


# speedup vs baseline: 5.5579x; 1.1386x over previous
import functools

import jax
import jax.numpy as jnp
from jax import lax
from jax.experimental import pallas as pl
from jax.experimental.pallas import tpu as pltpu
from jax.experimental.pallas import tpu_sc as plsc

N = 10000
E = 160000
D = 128
D_EDGE = 4
FC = 64
NUM_HEADS = 4
HEAD_DIM = 32
MUL_ALPHA = 32

NC = 2
NS = 16
NW = NC * NS
CHUNK = 40
NPAD = 10240
ROWS_PER_S = NPAD // NS

E0 = 64000
E1 = 96000

TE = 2000

N_BLK = 1000
GRID_N = N // N_BLK


def _sigmoid(x):
    return 1.0 / (1.0 + jnp.exp(-x))


def _silu(x):
    return x * _sigmoid(x)


def _smooth_leaky_relu(x, a=0.2):
    return 0.5 * (1.0 + a) * x + 0.5 * (1.0 - a) * x * (2.0 * _sigmoid(x) - 1.0)


def _node_kernel(x_ref, attr_ref, ws_ref, bs_ref, wd_ref, bd_ref, os_ref, od_ref):
    x = x_ref[...] * attr_ref[...]
    os_ref[...] = jnp.dot(x, ws_ref[...], preferred_element_type=jnp.float32) + bs_ref[...]
    od_ref[...] = jnp.dot(x, wd_ref[...], preferred_element_type=jnp.float32) + bd_ref[...]


def _node_messages(node_input, node_attr, W_src, b_src, W_dst, b_dst):
    return pl.pallas_call(
        _node_kernel,
        grid=(GRID_N,),
        in_specs=[
            pl.BlockSpec((N_BLK, D), lambda i: (i, 0)),
            pl.BlockSpec((N_BLK, 1), lambda i: (i, 0)),
            pl.BlockSpec((D, D), lambda i: (0, 0)),
            pl.BlockSpec((1, D), lambda i: (0, 0)),
            pl.BlockSpec((D, D), lambda i: (0, 0)),
            pl.BlockSpec((1, D), lambda i: (0, 0)),
        ],
        out_specs=[
            pl.BlockSpec((N_BLK, D), lambda i: (i, 0)),
            pl.BlockSpec((N_BLK, D), lambda i: (i, 0)),
        ],
        out_shape=[
            jax.ShapeDtypeStruct((N, D), jnp.float32),
            jax.ShapeDtypeStruct((N, D), jnp.float32),
        ],
    )(node_input, node_attr, W_src, b_src.reshape(1, D), W_dst, b_dst.reshape(1, D))


def _make_gather(e_sz):
    cpw = e_sz // (NW * CHUNK)
    quads, tail = divmod(cpw, 4)

    def body(tbl_hbm, idx_hbm, out_hbm, idx_v, b0, b1, b2, b3, s0, s1, s2, s3):
        c = lax.axis_index("c")
        s = lax.axis_index("s")
        wid = c * NS + s
        pltpu.sync_copy(idx_hbm.at[wid], idx_v)
        base = wid * cpw
        bufs = (b0, b1, b2, b3)
        sems = (s0, s1, s2, s3)

        def quad(q, carry):
            g = q * 4
            cps = [pltpu.async_copy(tbl_hbm.at[idx_v.at[g + k]], bufs[k], sems[k])
                   for k in range(4)]
            for k in range(4):
                cps[k].wait()
                pltpu.sync_copy(
                    bufs[k], out_hbm.at[pl.ds((base + g + k) * CHUNK, CHUNK)])
            return carry

        lax.fori_loop(0, quads, quad, 0)
        g0 = quads * 4
        tcps = [pltpu.async_copy(tbl_hbm.at[idx_v.at[g0 + k]], bufs[k], sems[k])
                for k in range(tail)]
        for k in range(tail):
            tcps[k].wait()
            pltpu.sync_copy(bufs[k],
                            out_hbm.at[pl.ds((base + g0 + k) * CHUNK, CHUNK)])

    return functools.partial(
        pl.kernel,
        out_type=jax.ShapeDtypeStruct((e_sz, D), jnp.float32),
        mesh=plsc.VectorSubcoreMesh(core_axis_name="c", subcore_axis_name="s"),
        scratch_types=[
            pltpu.VMEM((cpw, CHUNK), jnp.int32),
            pltpu.VMEM((CHUNK, D), jnp.float32),
            pltpu.VMEM((CHUNK, D), jnp.float32),
            pltpu.VMEM((CHUNK, D), jnp.float32),
            pltpu.VMEM((CHUNK, D), jnp.float32),
            pltpu.SemaphoreType.DMA,
            pltpu.SemaphoreType.DMA,
            pltpu.SemaphoreType.DMA,
            pltpu.SemaphoreType.DMA,
        ],
    )(body)


_sc_gather = {e: _make_gather(e) for e in (E0, E1)}


def _edge_kernel(gs_ref, gd_ref, sc_ref, ea_ref, wr0_ref, br0_ref, wr1_ref,
                 br1_ref, wr2_ref, br2_ref, wla_ref, wlv_ref, bla_ref, blv_ref,
                 a_ref, out_ref):
    h = _silu(jnp.dot(sc_ref[...], wr0_ref[...], preferred_element_type=jnp.float32)
              + br0_ref[...])
    h = _silu(jnp.dot(h, wr1_ref[...], preferred_element_type=jnp.float32)
              + br1_ref[...])
    msg = gs_ref[...] + gd_ref[...]
    ea = ea_ref[...]
    w2 = wr2_ref[...]
    b2 = br2_ref[...]
    wla = wla_ref[...]
    wlv = wlv_ref[...]
    f_a = jnp.broadcast_to(bla_ref[...], (TE, MUL_ALPHA))
    f_v = jnp.broadcast_to(blv_ref[...], (TE, D))
    for c in range(D_EDGE):
        w_c = (jnp.dot(h, w2[:, c * D:(c + 1) * D], preferred_element_type=jnp.float32)
               + b2[:, c * D:(c + 1) * D])
        d_c = msg * ea[:, c:c + 1] * w_c
        f_a = f_a + jnp.dot(d_c, wla[c * D:(c + 1) * D, :],
                            preferred_element_type=jnp.float32)
        f_v = f_v + jnp.dot(d_c, wlv[c * D:(c + 1) * D, :],
                            preferred_element_type=jnp.float32)
    alpha = _smooth_leaky_relu(f_a)
    logits = jnp.dot(alpha, a_ref[...], preferred_element_type=jnp.float32)
    ex = jnp.exp(logits)
    val = _silu(f_v)
    head = lax.broadcasted_iota(jnp.int32, (NUM_HEADS, D), 0)
    lane_head = lax.broadcasted_iota(jnp.int32, (NUM_HEADS, D), 1) // HEAD_DIM
    sel = (head == lane_head).astype(jnp.float32)
    exb = jnp.dot(ex, sel, preferred_element_type=jnp.float32)
    out_ref[0] = val * exb
    out_ref[1] = exb


def _edge_pipeline(g_src, g_dst, edge_scalars, edge_attr, W_r0, b_r0, W_r1,
                   b_r1, W_r2p, b_r2p, W_la, W_lv, b_la, b_lv, A):
    e_sz = g_src.shape[0]
    rep = lambda shape: pl.BlockSpec(shape, lambda i: tuple(0 for _ in shape))
    return pl.pallas_call(
        _edge_kernel,
        grid=(e_sz // TE,),
        in_specs=[
            pl.BlockSpec((TE, D), lambda i: (i, 0)),
            pl.BlockSpec((TE, D), lambda i: (i, 0)),
            pl.BlockSpec((TE, FC), lambda i: (i, 0)),
            pl.BlockSpec((TE, D_EDGE), lambda i: (i, 0)),
            rep((FC, FC)),
            rep((1, FC)),
            rep((FC, FC)),
            rep((1, FC)),
            rep((FC, D * D_EDGE)),
            rep((1, D * D_EDGE)),
            rep((D * D_EDGE, MUL_ALPHA)),
            rep((D * D_EDGE, D)),
            rep((1, MUL_ALPHA)),
            rep((1, D)),
            rep((MUL_ALPHA, NUM_HEADS)),
        ],
        out_specs=pl.BlockSpec((2, TE, D), lambda i: (0, i, 0)),
        out_shape=jax.ShapeDtypeStruct((2, e_sz, D), jnp.float32),
    )(g_src, g_dst, edge_scalars, edge_attr, W_r0, b_r0.reshape(1, FC), W_r1,
      b_r1.reshape(1, FC), W_r2p, b_r2p.reshape(1, D * D_EDGE), W_la, W_lv,
      b_la.reshape(1, MUL_ALPHA), b_lv.reshape(1, D), A)


def _make_scatter(e_sz):
    cps = e_sz // (NS * CHUNK)

    def body(attn_hbm, idx_hbm, zeros_hbm, out_hbm, idx_v, b0, b1, s0, s1,
             acc_sh):
        c = lax.axis_index("c")
        s = lax.axis_index("s")
        pltpu.sync_copy(zeros_hbm.at[pl.ds(s * ROWS_PER_S, ROWS_PER_S)],
                        acc_sh.at[pl.ds(s * ROWS_PER_S, ROWS_PER_S)])
        pltpu.sync_copy(idx_hbm.at[s], idx_v)
        plsc.subcore_barrier()
        base = s * cps

        def pair(p, carry):
            g = p * 2
            cp0 = pltpu.async_copy(
                attn_hbm.at[c, pl.ds((base + g) * CHUNK, CHUNK)], b0, s0)
            cp1 = pltpu.async_copy(
                attn_hbm.at[c, pl.ds((base + g + 1) * CHUNK, CHUNK)], b1, s1)
            cp0.wait()
            pltpu.sync_copy(b0, acc_sh.at[idx_v.at[g]], add=True)
            cp1.wait()
            pltpu.sync_copy(b1, acc_sh.at[idx_v.at[g + 1]], add=True)
            return carry

        lax.fori_loop(0, cps // 2, pair, 0)
        plsc.subcore_barrier()
        pltpu.sync_copy(acc_sh.at[pl.ds(s * ROWS_PER_S, ROWS_PER_S)],
                        out_hbm.at[c, pl.ds(s * ROWS_PER_S, ROWS_PER_S)])

    return functools.partial(
        pl.kernel,
        out_type=jax.ShapeDtypeStruct((NC, NPAD, D), jnp.float32),
        mesh=plsc.VectorSubcoreMesh(core_axis_name="c", subcore_axis_name="s"),
        scratch_types=[
            pltpu.VMEM((cps, CHUNK), jnp.int32),
            pltpu.VMEM((CHUNK, D), jnp.float32),
            pltpu.VMEM((CHUNK, D), jnp.float32),
            pltpu.SemaphoreType.DMA,
            pltpu.SemaphoreType.DMA,
            pltpu.VMEM_SHARED((NPAD, D), jnp.float32),
        ],
    )(body)


_sc_scatter = {e: _make_scatter(e) for e in (E0, E1)}


def _proj_kernel(a0_ref, a1_ref, b0_ref, b1_ref, wp_ref, bp_ref, o_ref):
    num = a0_ref[0] + b0_ref[0]
    den = a1_ref[0] + b1_ref[0]
    x = num / (den + 1e-9)
    o_ref[...] = (jnp.dot(x, wp_ref[...], preferred_element_type=jnp.float32)
                  + bp_ref[...])


def _project(acc_a, acc_b, W_proj, b_proj):
    return pl.pallas_call(
        _proj_kernel,
        grid=(GRID_N,),
        in_specs=[
            pl.BlockSpec((1, N_BLK, D), lambda i: (0, i, 0)),
            pl.BlockSpec((1, N_BLK, D), lambda i: (1, i, 0)),
            pl.BlockSpec((1, N_BLK, D), lambda i: (0, i, 0)),
            pl.BlockSpec((1, N_BLK, D), lambda i: (1, i, 0)),
            pl.BlockSpec((D, D), lambda i: (0, 0)),
            pl.BlockSpec((1, D), lambda i: (0, 0)),
        ],
        out_specs=pl.BlockSpec((N_BLK, D), lambda i: (i, 0)),
        out_shape=jax.ShapeDtypeStruct((N, D), jnp.float32),
    )(acc_a, acc_a, acc_b, acc_b, W_proj, b_proj.reshape(1, D))


def kernel(node_input, node_attr, edge_src, edge_dst, edge_attr, edge_scalars,
           batch, W_src, b_src, W_dst, b_dst, W_r0, b_r0, W_r1, b_r1, W_r2,
           b_r2, W_lin, b_lin, alpha_dot, W_proj, b_proj):
    W_r2p = W_r2.reshape(FC, D, D_EDGE).transpose(0, 2, 1).reshape(FC, D * D_EDGE)
    b_r2p = b_r2.reshape(D, D_EDGE).T.reshape(D * D_EDGE)
    W_linp = W_lin.reshape(D, D_EDGE, MUL_ALPHA + D).transpose(1, 0, 2)
    W_linp = W_linp.reshape(D * D_EDGE, MUL_ALPHA + D)
    W_la = W_linp[:, :MUL_ALPHA]
    W_lv = W_linp[:, MUL_ALPHA:]
    b_la = b_lin[:MUL_ALPHA]
    b_lv = b_lin[MUL_ALPHA:]
    flat = alpha_dot.reshape(-1)
    rows = jnp.arange(MUL_ALPHA)
    A = jnp.zeros((MUL_ALPHA, NUM_HEADS), jnp.float32).at[rows, rows // 8].set(flat)

    src32 = edge_src.astype(jnp.int32)
    dst32 = edge_dst.astype(jnp.int32)
    msg_src, msg_dst = _node_messages(node_input, node_attr, W_src, b_src,
                                      W_dst, b_dst)
    zeros = jnp.zeros((NPAD, D), jnp.float32)

    accs = []
    off = 0
    for e_sz in (E0, E1):
        sl = slice(off, off + e_sz)
        src_idx = src32[sl].reshape(NW, e_sz // (NW * CHUNK), CHUNK)
        dst_idx = dst32[sl].reshape(NW, e_sz // (NW * CHUNK), CHUNK)
        dst_idx_sc = dst32[sl].reshape(NS, e_sz // (NS * CHUNK), CHUNK)
        g_src = _sc_gather[e_sz](msg_src, src_idx)
        g_dst = _sc_gather[e_sz](msg_dst, dst_idx)
        attn_ext = _edge_pipeline(g_src, g_dst, edge_scalars[sl],
                                  edge_attr[sl], W_r0, b_r0, W_r1, b_r1,
                                  W_r2p, b_r2p, W_la, W_lv, b_la, b_lv, A)
        accs.append(_sc_scatter[e_sz](attn_ext, dst_idx_sc, zeros))
        off += e_sz

    return _project(accs[0], accs[1], W_proj, b_proj)


# speedup vs baseline: 5.6150x; 1.0103x over previous
import functools

import jax
import jax.numpy as jnp
from jax import lax
from jax.experimental import pallas as pl
from jax.experimental.pallas import tpu as pltpu
from jax.experimental.pallas import tpu_sc as plsc

N = 10000
E = 160000
D = 128
D_EDGE = 4
FC = 64
NUM_HEADS = 4
HEAD_DIM = 32
MUL_ALPHA = 32

NC = 2
NS = 16
NW = NC * NS
CHUNK = 40
NPAD = 10240
ROWS_PER_S = NPAD // NS

SLICES = (32000, 64000, 64000)

TE = 2000

N_BLK = 1000
GRID_N = N // N_BLK


def _sigmoid(x):
    return 1.0 / (1.0 + jnp.exp(-x))


def _silu(x):
    return x * _sigmoid(x)


def _smooth_leaky_relu(x, a=0.2):
    return 0.5 * (1.0 + a) * x + 0.5 * (1.0 - a) * x * (2.0 * _sigmoid(x) - 1.0)


def _node_kernel(x_ref, attr_ref, ws_ref, bs_ref, wd_ref, bd_ref, os_ref, od_ref):
    x = x_ref[...] * attr_ref[...]
    os_ref[...] = jnp.dot(x, ws_ref[...], preferred_element_type=jnp.float32) + bs_ref[...]
    od_ref[...] = jnp.dot(x, wd_ref[...], preferred_element_type=jnp.float32) + bd_ref[...]


def _node_messages(node_input, node_attr, W_src, b_src, W_dst, b_dst):
    return pl.pallas_call(
        _node_kernel,
        grid=(GRID_N,),
        in_specs=[
            pl.BlockSpec((N_BLK, D), lambda i: (i, 0)),
            pl.BlockSpec((N_BLK, 1), lambda i: (i, 0)),
            pl.BlockSpec((D, D), lambda i: (0, 0)),
            pl.BlockSpec((1, D), lambda i: (0, 0)),
            pl.BlockSpec((D, D), lambda i: (0, 0)),
            pl.BlockSpec((1, D), lambda i: (0, 0)),
        ],
        out_specs=[
            pl.BlockSpec((N_BLK, D), lambda i: (i, 0)),
            pl.BlockSpec((N_BLK, D), lambda i: (i, 0)),
        ],
        out_shape=[
            jax.ShapeDtypeStruct((N, D), jnp.float32),
            jax.ShapeDtypeStruct((N, D), jnp.float32),
        ],
    )(node_input, node_attr, W_src, b_src.reshape(1, D), W_dst, b_dst.reshape(1, D))


def _make_gather(e_sz):
    cpw = e_sz // (NW * CHUNK)
    quads, tail = divmod(cpw, 4)

    def body(tbl_hbm, idx_hbm, out_hbm, idx_v, b0, b1, b2, b3, s0, s1, s2, s3):
        c = lax.axis_index("c")
        s = lax.axis_index("s")
        wid = c * NS + s
        pltpu.sync_copy(idx_hbm.at[wid], idx_v)
        base = wid * cpw
        bufs = (b0, b1, b2, b3)
        sems = (s0, s1, s2, s3)

        def quad(q, carry):
            g = q * 4
            cps = [pltpu.async_copy(tbl_hbm.at[idx_v.at[g + k]], bufs[k], sems[k])
                   for k in range(4)]
            for k in range(4):
                cps[k].wait()
                pltpu.sync_copy(
                    bufs[k], out_hbm.at[pl.ds((base + g + k) * CHUNK, CHUNK)])
            return carry

        lax.fori_loop(0, quads, quad, 0)
        g0 = quads * 4
        tcps = [pltpu.async_copy(tbl_hbm.at[idx_v.at[g0 + k]], bufs[k], sems[k])
                for k in range(tail)]
        for k in range(tail):
            tcps[k].wait()
            pltpu.sync_copy(bufs[k],
                            out_hbm.at[pl.ds((base + g0 + k) * CHUNK, CHUNK)])

    return functools.partial(
        pl.kernel,
        out_type=jax.ShapeDtypeStruct((e_sz, D), jnp.float32),
        mesh=plsc.VectorSubcoreMesh(core_axis_name="c", subcore_axis_name="s"),
        scratch_types=[
            pltpu.VMEM((cpw, CHUNK), jnp.int32),
            pltpu.VMEM((CHUNK, D), jnp.float32),
            pltpu.VMEM((CHUNK, D), jnp.float32),
            pltpu.VMEM((CHUNK, D), jnp.float32),
            pltpu.VMEM((CHUNK, D), jnp.float32),
            pltpu.SemaphoreType.DMA,
            pltpu.SemaphoreType.DMA,
            pltpu.SemaphoreType.DMA,
            pltpu.SemaphoreType.DMA,
        ],
    )(body)


_sc_gather = {e: _make_gather(e) for e in set(SLICES)}


def _edge_kernel(gs_ref, gd_ref, sc_ref, ea_ref, wr0_ref, br0_ref, wr1_ref,
                 br1_ref, wr2_ref, br2_ref, wla_ref, wlv_ref, bla_ref, blv_ref,
                 a_ref, out_ref):
    h = _silu(jnp.dot(sc_ref[...], wr0_ref[...], preferred_element_type=jnp.float32)
              + br0_ref[...])
    h = _silu(jnp.dot(h, wr1_ref[...], preferred_element_type=jnp.float32)
              + br1_ref[...])
    msg = gs_ref[...] + gd_ref[...]
    ea = ea_ref[...]
    w2 = wr2_ref[...]
    b2 = br2_ref[...]
    wla = wla_ref[...]
    wlv = wlv_ref[...]
    f_a = jnp.broadcast_to(bla_ref[...], (TE, MUL_ALPHA))
    f_v = jnp.broadcast_to(blv_ref[...], (TE, D))
    for c in range(D_EDGE):
        w_c = (jnp.dot(h, w2[:, c * D:(c + 1) * D], preferred_element_type=jnp.float32)
               + b2[:, c * D:(c + 1) * D])
        d_c = msg * ea[:, c:c + 1] * w_c
        f_a = f_a + jnp.dot(d_c, wla[c * D:(c + 1) * D, :],
                            preferred_element_type=jnp.float32)
        f_v = f_v + jnp.dot(d_c, wlv[c * D:(c + 1) * D, :],
                            preferred_element_type=jnp.float32)
    alpha = _smooth_leaky_relu(f_a)
    logits = jnp.dot(alpha, a_ref[...], preferred_element_type=jnp.float32)
    ex = jnp.exp(logits)
    val = _silu(f_v)
    head = lax.broadcasted_iota(jnp.int32, (NUM_HEADS, D), 0)
    lane_head = lax.broadcasted_iota(jnp.int32, (NUM_HEADS, D), 1) // HEAD_DIM
    sel = (head == lane_head).astype(jnp.float32)
    exb = jnp.dot(ex, sel, preferred_element_type=jnp.float32)
    out_ref[0] = val * exb
    out_ref[1] = exb


def _edge_pipeline(g_src, g_dst, edge_scalars, edge_attr, W_r0, b_r0, W_r1,
                   b_r1, W_r2p, b_r2p, W_la, W_lv, b_la, b_lv, A):
    e_sz = g_src.shape[0]
    rep = lambda shape: pl.BlockSpec(shape, lambda i: tuple(0 for _ in shape))
    return pl.pallas_call(
        _edge_kernel,
        grid=(e_sz // TE,),
        in_specs=[
            pl.BlockSpec((TE, D), lambda i: (i, 0)),
            pl.BlockSpec((TE, D), lambda i: (i, 0)),
            pl.BlockSpec((TE, FC), lambda i: (i, 0)),
            pl.BlockSpec((TE, D_EDGE), lambda i: (i, 0)),
            rep((FC, FC)),
            rep((1, FC)),
            rep((FC, FC)),
            rep((1, FC)),
            rep((FC, D * D_EDGE)),
            rep((1, D * D_EDGE)),
            rep((D * D_EDGE, MUL_ALPHA)),
            rep((D * D_EDGE, D)),
            rep((1, MUL_ALPHA)),
            rep((1, D)),
            rep((MUL_ALPHA, NUM_HEADS)),
        ],
        out_specs=pl.BlockSpec((2, TE, D), lambda i: (0, i, 0)),
        out_shape=jax.ShapeDtypeStruct((2, e_sz, D), jnp.float32),
    )(g_src, g_dst, edge_scalars, edge_attr, W_r0, b_r0.reshape(1, FC), W_r1,
      b_r1.reshape(1, FC), W_r2p, b_r2p.reshape(1, D * D_EDGE), W_la, W_lv,
      b_la.reshape(1, MUL_ALPHA), b_lv.reshape(1, D), A)


def _make_scatter(e_sz):
    cps = e_sz // (NS * CHUNK)

    def body(attn_hbm, idx_hbm, zeros_hbm, out_hbm, idx_v, b0, b1, s0, s1,
             acc_sh):
        c = lax.axis_index("c")
        s = lax.axis_index("s")
        pltpu.sync_copy(zeros_hbm.at[pl.ds(s * ROWS_PER_S, ROWS_PER_S)],
                        acc_sh.at[pl.ds(s * ROWS_PER_S, ROWS_PER_S)])
        pltpu.sync_copy(idx_hbm.at[s], idx_v)
        plsc.subcore_barrier()
        base = s * cps

        def pair(p, carry):
            g = p * 2
            cp0 = pltpu.async_copy(
                attn_hbm.at[c, pl.ds((base + g) * CHUNK, CHUNK)], b0, s0)
            cp1 = pltpu.async_copy(
                attn_hbm.at[c, pl.ds((base + g + 1) * CHUNK, CHUNK)], b1, s1)
            cp0.wait()
            pltpu.sync_copy(b0, acc_sh.at[idx_v.at[g]], add=True)
            cp1.wait()
            pltpu.sync_copy(b1, acc_sh.at[idx_v.at[g + 1]], add=True)
            return carry

        lax.fori_loop(0, cps // 2, pair, 0)
        plsc.subcore_barrier()
        pltpu.sync_copy(acc_sh.at[pl.ds(s * ROWS_PER_S, ROWS_PER_S)],
                        out_hbm.at[c, pl.ds(s * ROWS_PER_S, ROWS_PER_S)])

    return functools.partial(
        pl.kernel,
        out_type=jax.ShapeDtypeStruct((NC, NPAD, D), jnp.float32),
        mesh=plsc.VectorSubcoreMesh(core_axis_name="c", subcore_axis_name="s"),
        scratch_types=[
            pltpu.VMEM((cps, CHUNK), jnp.int32),
            pltpu.VMEM((CHUNK, D), jnp.float32),
            pltpu.VMEM((CHUNK, D), jnp.float32),
            pltpu.SemaphoreType.DMA,
            pltpu.SemaphoreType.DMA,
            pltpu.VMEM_SHARED((NPAD, D), jnp.float32),
        ],
    )(body)


_sc_scatter = {e: _make_scatter(e) for e in set(SLICES)}


def _proj_kernel(a0_ref, a1_ref, b0_ref, b1_ref, c0_ref, c1_ref, wp_ref,
                 bp_ref, o_ref):
    num = a0_ref[0] + b0_ref[0] + c0_ref[0]
    den = a1_ref[0] + b1_ref[0] + c1_ref[0]
    x = num / (den + 1e-9)
    o_ref[...] = (jnp.dot(x, wp_ref[...], preferred_element_type=jnp.float32)
                  + bp_ref[...])


def _project(acc_a, acc_b, acc_c, W_proj, b_proj):
    pspec = lambda c: pl.BlockSpec((1, N_BLK, D), lambda i, c=c: (c, i, 0))
    return pl.pallas_call(
        _proj_kernel,
        grid=(GRID_N,),
        in_specs=[
            pspec(0), pspec(1), pspec(0), pspec(1), pspec(0), pspec(1),
            pl.BlockSpec((D, D), lambda i: (0, 0)),
            pl.BlockSpec((1, D), lambda i: (0, 0)),
        ],
        out_specs=pl.BlockSpec((N_BLK, D), lambda i: (i, 0)),
        out_shape=jax.ShapeDtypeStruct((N, D), jnp.float32),
    )(acc_a, acc_a, acc_b, acc_b, acc_c, acc_c, W_proj, b_proj.reshape(1, D))


def kernel(node_input, node_attr, edge_src, edge_dst, edge_attr, edge_scalars,
           batch, W_src, b_src, W_dst, b_dst, W_r0, b_r0, W_r1, b_r1, W_r2,
           b_r2, W_lin, b_lin, alpha_dot, W_proj, b_proj):
    W_r2p = W_r2.reshape(FC, D, D_EDGE).transpose(0, 2, 1).reshape(FC, D * D_EDGE)
    b_r2p = b_r2.reshape(D, D_EDGE).T.reshape(D * D_EDGE)
    W_linp = W_lin.reshape(D, D_EDGE, MUL_ALPHA + D).transpose(1, 0, 2)
    W_linp = W_linp.reshape(D * D_EDGE, MUL_ALPHA + D)
    W_la = W_linp[:, :MUL_ALPHA]
    W_lv = W_linp[:, MUL_ALPHA:]
    b_la = b_lin[:MUL_ALPHA]
    b_lv = b_lin[MUL_ALPHA:]
    flat = alpha_dot.reshape(-1)
    rows = jnp.arange(MUL_ALPHA)
    A = jnp.zeros((MUL_ALPHA, NUM_HEADS), jnp.float32).at[rows, rows // 8].set(flat)

    src32 = edge_src.astype(jnp.int32)
    dst32 = edge_dst.astype(jnp.int32)
    msg_src, msg_dst = _node_messages(node_input, node_attr, W_src, b_src,
                                      W_dst, b_dst)
    zeros = jnp.zeros((NPAD, D), jnp.float32)

    accs = []
    off = 0
    for e_sz in SLICES:
        sl = slice(off, off + e_sz)
        src_idx = src32[sl].reshape(NW, e_sz // (NW * CHUNK), CHUNK)
        dst_idx = dst32[sl].reshape(NW, e_sz // (NW * CHUNK), CHUNK)
        dst_idx_sc = dst32[sl].reshape(NS, e_sz // (NS * CHUNK), CHUNK)
        g_src = _sc_gather[e_sz](msg_src, src_idx)
        g_dst = _sc_gather[e_sz](msg_dst, dst_idx)
        attn_ext = _edge_pipeline(g_src, g_dst, edge_scalars[sl],
                                  edge_attr[sl], W_r0, b_r0, W_r1, b_r1,
                                  W_r2p, b_r2p, W_la, W_lv, b_la, b_lv, A)
        accs.append(_sc_scatter[e_sz](attn_ext, dst_idx_sc, zeros))
        off += e_sz

    return _project(accs[0], accs[1], accs[2], W_proj, b_proj)
